# baseline (speedup 1.0000x reference)
import jax
import jax.numpy as jnp
from jax import lax
from jax.experimental import pallas as pl
from jax.experimental.pallas import tpu as pltpu
from jax.experimental.pallas import tpu_sc as plsc

N = 10000
D = 128
E = 320000
NC = 2
NS = 16
NW = NC * NS
NCH = E // 128
NBUF = 3
LOOK = 1
RING = 6
CPT = 78
XTRA = NCH - NW * CPT
RPT = 632
LAST = N - (NS - 1) * RPT
RPT_D = 640
LAST_D = N - (NS - 1) * RPT_D


def _mesh():
    return plsc.VectorSubcoreMesh(core_axis_name="c", subcore_axis_name="s")


def _zero_or_flush(sid, src, dst, rpt=RPT, last=LAST, align=8):
    r0 = pl.multiple_of(sid * rpt, align)

    @pl.when(sid < NS - 1)
    def _():
        pltpu.sync_copy(src.at[pl.ds(r0, rpt)], dst.at[pl.ds(r0, rpt)])

    @pl.when(sid == NS - 1)
    def _():
        pltpu.sync_copy(src.at[pl.ds(r0, last)], dst.at[pl.ds(r0, last)])


def _deg_body(ei_hbm, ones_hbm, zeros_hbm, out_hbm, ring, ones_v, accum,
              *sems):
    ss = sems[:NBUF]
    si = sems[NBUF:]
    cid = lax.axis_index("c")
    sid = lax.axis_index("s")
    wid = sid * NC + cid
    c0 = wid * CPT
    _zero_or_flush(sid, zeros_hbm, accum)
    pltpu.sync_copy(ones_hbm, ones_v)

    def fetch_idx(k, r):
        pltpu.async_copy(
            ei_hbm.at[1, pl.ds((c0 + k) * 128, 128)], ring.at[r], si[r])

    def wait_idx(r):
        pltpu.make_async_copy(
            ei_hbm.at[1, pl.ds(0, 128)], ring.at[r], si[r]).wait()

    for k in range(2):
        fetch_idx(k, k)
    plsc.subcore_barrier()

    def grp(g, c):
        for u in range(RING):
            j = g * RING + u
            b = u % NBUF

            @pl.when(j >= NBUF)
            def _():
                pltpu.make_async_copy(
                    ones_v, accum.at[ring.at[u]], ss[b]).wait()

            wait_idx(u)
            pltpu.async_copy(ones_v, accum.at[ring.at[u]], ss[b], add=True)
            fj = j + 2
            fu = (u + 2) % RING

            @pl.when(fj < CPT)
            def _():
                fetch_idx(fj, fu)
        return c

    lax.fori_loop(0, CPT // RING, grp, 0)
    for i in range(NBUF):
        j = CPT - NBUF + i
        pltpu.make_async_copy(
            ones_v, accum.at[ring.at[j % RING]], ss[i % NBUF]).wait()

    @pl.when(wid < XTRA)
    def _():
        fetch_idx(NW * CPT - c0 + wid, 0)
        wait_idx(0)
        pltpu.async_copy(ones_v, accum.at[ring.at[0]], ss[0], add=True)
        pltpu.make_async_copy(ones_v, accum.at[ring.at[0]], ss[0]).wait()

    plsc.subcore_barrier()
    _zero_or_flush(sid, accum, out_hbm.at[cid])


def _sc_degree(ei1d, ones2d, zeros2d):
    kern = pl.kernel(
        _deg_body,
        out_type=jax.ShapeDtypeStruct((NC, N, D), jnp.float32),
        mesh=_mesh(),
        scratch_types=[
            pltpu.VMEM((RING, 128), jnp.int32),
            pltpu.VMEM((128, D), jnp.float32),
            pltpu.VMEM_SHARED((N, D), jnp.float32),
        ] + [pltpu.SemaphoreType.DMA] * (NBUF + RING),
    )
    return kern(ei1d, ones2d, zeros2d)


def _scatter_body(h_hbm, ei_hbm, zeros_hbm, out_hbm, ring, *bufs):
    rows = bufs[:NBUF]
    accum = bufs[NBUF]
    sg = bufs[NBUF + 1:NBUF + 1 + NBUF]
    ss = bufs[NBUF + 1 + NBUF:NBUF + 1 + 2 * NBUF]
    si = bufs[NBUF + 1 + 2 * NBUF:]
    cid = lax.axis_index("c")
    sid = lax.axis_index("s")
    wid = sid * NC + cid
    c0 = wid * CPT
    _zero_or_flush(sid, zeros_hbm, accum)

    def fetch_idx(k, r):
        pltpu.async_copy(
            ei_hbm.at[0, pl.ds((c0 + k) * 128, 128)], ring.at[r, 0], si[r])
        pltpu.async_copy(
            ei_hbm.at[1, pl.ds((c0 + k) * 128, 128)], ring.at[r, 1], si[r])

    def wait_idx(r):
        pltpu.make_async_copy(
            ei_hbm.at[0, pl.ds(0, 128)], ring.at[r, 0], si[r]).wait()
        pltpu.make_async_copy(
            ei_hbm.at[1, pl.ds(0, 128)], ring.at[r, 1], si[r]).wait()

    def start_gather(r, b):
        pltpu.async_copy(h_hbm.at[ring.at[r, 0]], rows[b], sg[b])

    def wait_gather(r, b):
        pltpu.make_async_copy(h_hbm.at[ring.at[r, 0]], rows[b], sg[b]).wait()

    def start_scatter(r, b):
        pltpu.async_copy(rows[b], accum.at[ring.at[r, 1]], ss[b], add=True)

    def wait_scatter(r, b):
        pltpu.make_async_copy(rows[b], accum.at[ring.at[r, 1]], ss[b]).wait()

    for k in range(LOOK + 2):
        fetch_idx(k, k)
    plsc.subcore_barrier()
    for k in range(LOOK):
        wait_idx(k)
        start_gather(k, k)

    def grp(g, c):
        for u in range(RING):
            j = g * RING + u
            b = u % NBUF
            wait_gather(u, b)
            start_scatter(u, b)
            nj = j + LOOK
            nu = (u + LOOK) % RING
            nb = (u + LOOK) % NBUF

            @pl.when(nj < CPT)
            def _():
                @pl.when(nj >= NBUF)
                def _():
                    wait_scatter(nu, nb)

                wait_idx(nu)
                start_gather(nu, nb)

            fj = j + LOOK + 2
            fu = (u + LOOK + 2) % RING

            @pl.when(fj < CPT)
            def _():
                fetch_idx(fj, fu)
        return c

    lax.fori_loop(0, CPT // RING, grp, 0)
    for i in range(NBUF):
        j = CPT - NBUF + i
        wait_scatter(j % RING, j % NBUF)

    @pl.when(wid < XTRA)
    def _():
        fetch_idx(NW * CPT - c0 + wid, 0)
        wait_idx(0)
        start_gather(0, 0)
        wait_gather(0, 0)
        start_scatter(0, 0)
        wait_scatter(0, 0)

    plsc.subcore_barrier()
    _zero_or_flush(sid, accum, out_hbm.at[cid])


def _sc_scatter(hp, ei1d, zeros2d):
    kern = pl.kernel(
        _scatter_body,
        out_type=jax.ShapeDtypeStruct((NC, N, D), jnp.float32),
        mesh=_mesh(),
        scratch_types=[
            pltpu.VMEM((RING, 2, 128), jnp.int32),
        ] + [pltpu.VMEM((128, D), jnp.float32)] * NBUF + [
            pltpu.VMEM_SHARED((N, D), jnp.float32),
        ] + [pltpu.SemaphoreType.DMA] * (2 * NBUF + RING),
    )
    return kern(hp, ei1d, zeros2d)


def _mm_body(x_ref, w_ref, o_ref):
    o_ref[...] = jnp.dot(
        x_ref[...], w_ref[...], preferred_element_type=jnp.float32)


def _tc_matmul(x, W1):
    return pl.pallas_call(
        _mm_body,
        out_shape=jax.ShapeDtypeStruct((N, D), jnp.float32),
    )(x, W1)


def _stage0_body(degp_ref, hraw_ref, dinv_ref, h_ref):
    deg = degp_ref[0, :, 0:1] + degp_ref[1, :, 0:1] + 1.0
    dinv = lax.rsqrt(deg)
    dinv_ref[...] = dinv
    h_ref[...] = hraw_ref[...] * dinv


def _tc_stage0(deg_parts, hraw):
    return pl.pallas_call(
        _stage0_body,
        out_shape=(
            jax.ShapeDtypeStruct((N, 1), jnp.float32),
            jax.ShapeDtypeStruct((N, D), jnp.float32),
        ),
    )(deg_parts, hraw)


def _mid_body(p_ref, h_ref, dinv_ref, b_ref, g_ref, bt_ref, w_ref, o_ref):
    agg = p_ref[0] + p_ref[1] + h_ref[...]
    conv = dinv_ref[...] * agg + b_ref[...][None, :]
    m = jnp.sum(conv, axis=0, keepdims=True) * (1.0 / N)
    cc = conv - m
    v = jnp.sum(cc * cc, axis=0, keepdims=True) * (1.0 / N)
    t = cc * lax.rsqrt(v + 1e-5) * g_ref[...][None, :] + bt_ref[...][None, :]
    t = jnp.maximum(t, 0.0)
    o_ref[...] = dinv_ref[...] * jnp.dot(
        t, w_ref[...], preferred_element_type=jnp.float32)


def _tc_mid(parts, hp, dinv, b, g, bt, Wn):
    return pl.pallas_call(
        _mid_body,
        out_shape=jax.ShapeDtypeStruct((N, D), jnp.float32),
    )(parts, hp, dinv, b, g, bt, Wn)


def _final_body(p_ref, h_ref, dinv_ref, b_ref, o_ref):
    agg = p_ref[0] + p_ref[1] + h_ref[...]
    o_ref[...] = dinv_ref[...] * agg + b_ref[...][None, :]


def _tc_final(parts, hp, dinv, b):
    return pl.pallas_call(
        _final_body,
        out_shape=jax.ShapeDtypeStruct((N, D), jnp.float32),
    )(parts, hp, dinv, b)


def kernel(x, edge_index, W1, b1, g1, bt1, W2, b2, g2, bt2, W3, b3):
    ei32 = edge_index.astype(jnp.int32)
    zeros2d = jnp.zeros((N, D), jnp.float32)
    ones2d = jnp.ones((128, D), jnp.float32)

    deg_parts = _sc_degree(ei32, ones2d, zeros2d)
    h1raw = _tc_matmul(x, W1)
    dinv, h1 = _tc_stage0(deg_parts, h1raw)
    p1 = _sc_scatter(h1, ei32, zeros2d)
    h2 = _tc_mid(p1, h1, dinv, b1, g1, bt1, W2)
    p2 = _sc_scatter(h2, ei32, zeros2d)
    h3 = _tc_mid(p2, h2, dinv, b2, g2, bt2, W3)
    p3 = _sc_scatter(h3, ei32, zeros2d)
    return _tc_final(p3, h3, dinv, b3)


# speedup vs baseline: 1.1898x; 1.1898x over previous
import jax
import jax.numpy as jnp
from jax import lax
from jax.experimental import pallas as pl
from jax.experimental.pallas import tpu as pltpu
from jax.experimental.pallas import tpu_sc as plsc

N = 10000
D = 128
E = 320000
NC = 2
NS = 16
NW = NC * NS
NCH = E // 128
NBUF = 3
LOOK = 2
RING = 6
CPT = 78
XTRA = NCH - NW * CPT
RPT = 632
LAST = N - (NS - 1) * RPT


def _mesh():
    return plsc.VectorSubcoreMesh(core_axis_name="c", subcore_axis_name="s")


def _zero_or_flush(sid, src, dst, rpt=RPT, last=LAST, align=8):
    r0 = pl.multiple_of(sid * rpt, align)

    @pl.when(sid < NS - 1)
    def _():
        pltpu.sync_copy(src.at[pl.ds(r0, rpt)], dst.at[pl.ds(r0, rpt)])

    @pl.when(sid == NS - 1)
    def _():
        pltpu.sync_copy(src.at[pl.ds(r0, last)], dst.at[pl.ds(r0, last)])


def _deg_body(ei_hbm, ones_hbm, zeros_hbm, out_hbm, ring, ones_v, accum,
              *sems):
    ss = sems[:NBUF]
    si = sems[NBUF:]
    cid = lax.axis_index("c")
    sid = lax.axis_index("s")
    wid = sid * NC + cid
    c0 = wid * CPT
    _zero_or_flush(sid, zeros_hbm, accum)
    pltpu.sync_copy(ones_hbm, ones_v)

    def fetch_idx(k, r):
        pltpu.async_copy(
            ei_hbm.at[1, pl.ds((c0 + k) * 128, 128)], ring.at[r], si[r])

    def wait_idx(r):
        pltpu.make_async_copy(
            ei_hbm.at[1, pl.ds(0, 128)], ring.at[r], si[r]).wait()

    for k in range(2):
        fetch_idx(k, k)
    plsc.subcore_barrier()

    def grp(g, c):
        for u in range(RING):
            j = g * RING + u
            b = u % NBUF

            @pl.when(j >= NBUF)
            def _():
                pltpu.make_async_copy(
                    ones_v, accum.at[ring.at[u]], ss[b]).wait()

            wait_idx(u)
            pltpu.async_copy(ones_v, accum.at[ring.at[u]], ss[b], add=True)
            fj = j + 2
            fu = (u + 2) % RING

            @pl.when(fj < CPT)
            def _():
                fetch_idx(fj, fu)
        return c

    lax.fori_loop(0, CPT // RING, grp, 0)
    for i in range(NBUF):
        j = CPT - NBUF + i
        pltpu.make_async_copy(
            ones_v, accum.at[ring.at[j % RING]], ss[i % NBUF]).wait()

    @pl.when(wid < XTRA)
    def _():
        fetch_idx(NW * CPT - c0 + wid, 0)
        wait_idx(0)
        pltpu.async_copy(ones_v, accum.at[ring.at[0]], ss[0], add=True)
        pltpu.make_async_copy(ones_v, accum.at[ring.at[0]], ss[0]).wait()

    plsc.subcore_barrier()
    _zero_or_flush(sid, accum, out_hbm.at[cid])


def _sc_degree(ei1d, ones2d, zeros2d):
    kern = pl.kernel(
        _deg_body,
        out_type=jax.ShapeDtypeStruct((NC, N, D), jnp.float32),
        mesh=_mesh(),
        scratch_types=[
            pltpu.VMEM((RING, 128), jnp.int32),
            pltpu.VMEM((128, D), jnp.float32),
            pltpu.VMEM_SHARED((N, D), jnp.float32),
        ] + [pltpu.SemaphoreType.DMA] * (NBUF + RING),
    )
    return kern(ei1d, ones2d, zeros2d)


def _scatter_body(h_hbm, ei_hbm, zeros_hbm, out_hbm, ring, *bufs):
    rows = bufs[:NBUF]
    accum = bufs[NBUF]
    sg = bufs[NBUF + 1:NBUF + 1 + NBUF]
    ss = bufs[NBUF + 1 + NBUF:NBUF + 1 + 2 * NBUF]
    si = bufs[NBUF + 1 + 2 * NBUF:]
    cid = lax.axis_index("c")
    sid = lax.axis_index("s")
    wid = sid * NC + cid
    c0 = wid * CPT
    _zero_or_flush(sid, zeros_hbm, accum)

    def fetch_idx(k, r):
        pltpu.async_copy(
            ei_hbm.at[0, pl.ds((c0 + k) * 128, 128)], ring.at[r, 0], si[r])
        pltpu.async_copy(
            ei_hbm.at[1, pl.ds((c0 + k) * 128, 128)], ring.at[r, 1], si[r])

    def wait_idx(r):
        pltpu.make_async_copy(
            ei_hbm.at[0, pl.ds(0, 128)], ring.at[r, 0], si[r]).wait()
        pltpu.make_async_copy(
            ei_hbm.at[1, pl.ds(0, 128)], ring.at[r, 1], si[r]).wait()

    def start_gather(r, b):
        pltpu.async_copy(h_hbm.at[ring.at[r, 0]], rows[b], sg[b])

    def wait_gather(r, b):
        pltpu.make_async_copy(h_hbm.at[ring.at[r, 0]], rows[b], sg[b]).wait()

    def start_scatter(r, b):
        pltpu.async_copy(rows[b], accum.at[ring.at[r, 1]], ss[b], add=True)

    def wait_scatter(r, b):
        pltpu.make_async_copy(rows[b], accum.at[ring.at[r, 1]], ss[b]).wait()

    for k in range(LOOK + 2):
        fetch_idx(k, k)
    plsc.subcore_barrier()
    for k in range(LOOK):
        wait_idx(k)
        start_gather(k, k)

    def grp(g, c):
        for u in range(RING):
            j = g * RING + u
            b = u % NBUF
            wait_gather(u, b)
            start_scatter(u, b)
            nj = j + LOOK
            nu = (u + LOOK) % RING
            nb = (u + LOOK) % NBUF

            @pl.when(nj < CPT)
            def _():
                @pl.when(nj >= NBUF)
                def _():
                    wait_scatter(nu, nb)

                wait_idx(nu)
                start_gather(nu, nb)

            fj = j + LOOK + 2
            fu = (u + LOOK + 2) % RING

            @pl.when(fj < CPT)
            def _():
                fetch_idx(fj, fu)
        return c

    lax.fori_loop(0, CPT // RING, grp, 0)
    for i in range(NBUF):
        j = CPT - NBUF + i
        wait_scatter(j % RING, j % NBUF)

    @pl.when(wid < XTRA)
    def _():
        fetch_idx(NW * CPT - c0 + wid, 0)
        wait_idx(0)
        start_gather(0, 0)
        wait_gather(0, 0)
        start_scatter(0, 0)
        wait_scatter(0, 0)

    plsc.subcore_barrier()
    _zero_or_flush(sid, accum, out_hbm.at[cid])


def _sc_scatter(hp, ei1d, zeros2d):
    kern = pl.kernel(
        _scatter_body,
        out_type=jax.ShapeDtypeStruct((NC, N, D), jnp.float32),
        mesh=_mesh(),
        scratch_types=[
            pltpu.VMEM((RING, 2, 128), jnp.int32),
        ] + [pltpu.VMEM((128, D), jnp.float32)] * NBUF + [
            pltpu.VMEM_SHARED((N, D), jnp.float32),
        ] + [pltpu.SemaphoreType.DMA] * (2 * NBUF + RING),
    )
    return kern(hp, ei1d, zeros2d)


def _mm_body(x_ref, w_ref, o_ref):
    o_ref[...] = jnp.dot(
        x_ref[...], w_ref[...], preferred_element_type=jnp.float32)


def _tc_matmul(x, W1):
    return pl.pallas_call(
        _mm_body,
        out_shape=jax.ShapeDtypeStruct((N, D), jnp.float32),
    )(x, W1)


def _stage0_body(degp_ref, hraw_ref, dinv_ref, h_ref):
    deg = degp_ref[0, :, 0:1] + degp_ref[1, :, 0:1] + 1.0
    dinv = lax.rsqrt(deg)
    dinv_ref[...] = dinv
    h_ref[...] = hraw_ref[...] * dinv


def _tc_stage0(deg_parts, hraw):
    return pl.pallas_call(
        _stage0_body,
        out_shape=(
            jax.ShapeDtypeStruct((N, 1), jnp.float32),
            jax.ShapeDtypeStruct((N, D), jnp.float32),
        ),
    )(deg_parts, hraw)


def _mid_body(p_ref, h_ref, dinv_ref, b_ref, g_ref, bt_ref, w_ref, o_ref):
    agg = p_ref[0] + p_ref[1] + h_ref[...]
    conv = dinv_ref[...] * agg + b_ref[...][None, :]
    m = jnp.sum(conv, axis=0, keepdims=True) * (1.0 / N)
    cc = conv - m
    v = jnp.sum(cc * cc, axis=0, keepdims=True) * (1.0 / N)
    t = cc * lax.rsqrt(v + 1e-5) * g_ref[...][None, :] + bt_ref[...][None, :]
    t = jnp.maximum(t, 0.0)
    o_ref[...] = dinv_ref[...] * jnp.dot(
        t, w_ref[...], preferred_element_type=jnp.float32)


def _tc_mid(parts, hp, dinv, b, g, bt, Wn):
    return pl.pallas_call(
        _mid_body,
        out_shape=jax.ShapeDtypeStruct((N, D), jnp.float32),
    )(parts, hp, dinv, b, g, bt, Wn)


def _final_body(p_ref, h_ref, dinv_ref, b_ref, o_ref):
    agg = p_ref[0] + p_ref[1] + h_ref[...]
    o_ref[...] = dinv_ref[...] * agg + b_ref[...][None, :]


def _tc_final(parts, hp, dinv, b):
    return pl.pallas_call(
        _final_body,
        out_shape=jax.ShapeDtypeStruct((N, D), jnp.float32),
    )(parts, hp, dinv, b)


def kernel(x, edge_index, W1, b1, g1, bt1, W2, b2, g2, bt2, W3, b3):
    ei32 = edge_index.astype(jnp.int32)
    zeros2d = jnp.zeros((N, D), jnp.float32)
    ones2d = jnp.ones((128, D), jnp.float32)

    deg_parts = _sc_degree(ei32, ones2d, zeros2d)
    h1raw = _tc_matmul(x, W1)
    dinv, h1 = _tc_stage0(deg_parts, h1raw)
    p1 = _sc_scatter(h1, ei32, zeros2d)
    h2 = _tc_mid(p1, h1, dinv, b1, g1, bt1, W2)
    p2 = _sc_scatter(h2, ei32, zeros2d)
    h3 = _tc_mid(p2, h2, dinv, b2, g2, bt2, W3)
    p3 = _sc_scatter(h3, ei32, zeros2d)
    return _tc_final(p3, h3, dinv, b3)
